# baseline (speedup 1.0000x reference)
import functools

import jax
import jax.numpy as jnp
from jax.experimental import pallas as pl
from jax.experimental.pallas import tpu as pltpu
from jax.experimental.pallas import tpu_sc as plsc

_BN = 128

_PREC = jax.lax.Precision.DEFAULT


def _b2_body(cbt_ref, b2_ref):
    cbt = cbt_ref[...]
    b2_ref[...] = jnp.sum(cbt * cbt, axis=0, keepdims=True)


def _codeword_norms(cbt):
    d, k = cbt.shape
    return pl.pallas_call(
        _b2_body,
        out_shape=jax.ShapeDtypeStruct((1, k), jnp.float32),
    )(cbt)


def _assign_body(x_ref, cbt_ref, b2_ref, key_ref, enc_ref):
    x = x_ref[...]
    a2 = jnp.sum(x * x, axis=1, keepdims=True)
    s = jax.lax.dot_general(
        x.astype(jnp.bfloat16), cbt_ref[...], (((1,), (0,)), ((), ())),
        preferred_element_type=jnp.float32, precision=_PREC,
    )
    d2 = (a2 - 2.0 * s) + b2_ref[...]
    m = jnp.min(d2, axis=1, keepdims=True)
    thr = jnp.where(m > 0, m * (1.0 + 0.9e-7), 0.0)
    keys = jnp.min(jnp.where(d2 <= thr, key_ref[...], 2.0), axis=1, keepdims=True)
    idx = jax.lax.bitcast_convert_type(keys, jnp.int32) - jnp.int32(0x3F800000)
    enc_ref[...] = idx.reshape(1, idx.shape[0])


def _assign(x, cbt_bf, b2, keys):
    n, d = x.shape
    k = cbt_bf.shape[1]
    return pl.pallas_call(
        _assign_body,
        grid=(n // _BN,),
        in_specs=[
            pl.BlockSpec((_BN, d), lambda i: (i, 0)),
            pl.BlockSpec((d, k), lambda i: (0, 0)),
            pl.BlockSpec((1, k), lambda i: (0, 0)),
            pl.BlockSpec((1, k), lambda i: (0, 0)),
        ],
        out_specs=pl.BlockSpec((1, _BN), lambda i: (0, i)),
        out_shape=jax.ShapeDtypeStruct((1, n), jnp.int32),
    )(x, cbt_bf, b2, keys)


def _sc_gather(codebook, idx2d):
    n = idx2d.shape[1]
    d = codebook.shape[1]
    mesh = plsc.VectorSubcoreMesh(core_axis_name="c", subcore_axis_name="s")
    units = mesh.num_cores * mesh.num_subcores
    per = n // units

    tile = 64
    nbuf = 4
    ntiles = per // tile

    @functools.partial(
        pl.kernel,
        out_type=jax.ShapeDtypeStruct((n, d), codebook.dtype),
        mesh=mesh,
        scratch_types=[
            pltpu.VMEM((per,), jnp.int32),
            pltpu.VMEM((nbuf, tile, d), jnp.float32),
            pltpu.SemaphoreType.DMA,
            pltpu.SemaphoreType.DMA((nbuf,)),
            pltpu.SemaphoreType.DMA((nbuf,)),
        ],
    )
    def _gather_kernel(cb_hbm, i_hbm, o_hbm, iv, buf, isem, gsems, osems):
        c = jax.lax.axis_index("c")
        s = jax.lax.axis_index("s")
        base = (c * mesh.num_subcores + s) * per
        pltpu.async_copy(i_hbm.at[0, pl.ds(base, per)], iv, isem).wait()
        g_cps, o_cps = {}, {}
        for t in range(ntiles):
            b = t % nbuf
            if t >= nbuf:
                o_cps[t - nbuf].wait()
            g_cps[t] = pltpu.async_copy(
                cb_hbm.at[iv.at[pl.ds(t * tile, tile)]], buf.at[b], gsems.at[b]
            )
            if t >= 1:
                tp = t - 1
                g_cps[tp].wait()
                o_cps[tp] = pltpu.async_copy(
                    buf.at[tp % nbuf],
                    o_hbm.at[pl.ds(base + tp * tile, tile)],
                    osems.at[tp % nbuf],
                )
        tl = ntiles - 1
        g_cps[tl].wait()
        o_cps[tl] = pltpu.async_copy(
            buf.at[tl % nbuf], o_hbm.at[pl.ds(base + tl * tile, tile)],
            osems.at[tl % nbuf],
        )
        for t in range(max(0, ntiles - nbuf), ntiles):
            o_cps[t].wait()

    return _gather_kernel(codebook, idx2d)


_CHUNKS = 1


def kernel(x, codebook):
    n = x.shape[0]
    cbt = codebook.T
    k = codebook.shape[0]
    b2 = _codeword_norms(cbt)
    cbt_bf = cbt.astype(jnp.bfloat16)
    keys = jax.lax.bitcast_convert_type(
        jnp.int32(0x3F800000) + jnp.arange(k, dtype=jnp.int32), jnp.float32
    ).reshape(1, k)
    encs = [_assign(xc, cbt_bf, b2, keys) for xc in jnp.split(x, _CHUNKS)]
    qs = [_sc_gather(codebook, enc) for enc in encs]
    encoded = jnp.concatenate(encs, axis=1).reshape(n)
    quantized = jnp.concatenate(qs)
    return (encoded, quantized)


# speedup vs baseline: 1.1778x; 1.1778x over previous
import functools

import jax
import jax.numpy as jnp
from jax.experimental import pallas as pl
from jax.experimental.pallas import tpu as pltpu
from jax.experimental.pallas import tpu_sc as plsc

_BN = 256

_PREC = jax.lax.Precision.DEFAULT


def _b2_body(cbt_ref, b2_ref):
    cbt = cbt_ref[...]
    b2_ref[...] = jnp.sum(cbt * cbt, axis=0, keepdims=True)


def _codeword_norms(cbt):
    d, k = cbt.shape
    return pl.pallas_call(
        _b2_body,
        out_shape=jax.ShapeDtypeStruct((1, k), jnp.float32),
    )(cbt)


def _assign_body(x_ref, cbt_ref, b2_ref, key_ref, enc_ref):
    x = x_ref[...]
    a2 = jnp.sum(x * x, axis=1, keepdims=True)
    s = jax.lax.dot_general(
        x.astype(jnp.bfloat16), cbt_ref[...], (((1,), (0,)), ((), ())),
        preferred_element_type=jnp.float32, precision=_PREC,
    )
    d2 = (a2 - 2.0 * s) + b2_ref[...]
    m = jnp.min(d2, axis=1, keepdims=True)
    thr = jnp.where(m > 0, m * (1.0 + 0.9e-7), 0.0)
    keys = jnp.min(jnp.where(d2 <= thr, key_ref[...], 2.0), axis=1, keepdims=True)
    idx = jax.lax.bitcast_convert_type(keys, jnp.int32) - jnp.int32(0x3F800000)
    enc_ref[...] = idx.reshape(1, idx.shape[0])


def _assign(x, cbt_bf, b2, keys):
    n, d = x.shape
    k = cbt_bf.shape[1]
    return pl.pallas_call(
        _assign_body,
        grid=(n // _BN,),
        in_specs=[
            pl.BlockSpec((_BN, d), lambda i: (i, 0)),
            pl.BlockSpec((d, k), lambda i: (0, 0)),
            pl.BlockSpec((1, k), lambda i: (0, 0)),
            pl.BlockSpec((1, k), lambda i: (0, 0)),
        ],
        out_specs=pl.BlockSpec((1, _BN), lambda i: (0, i)),
        out_shape=jax.ShapeDtypeStruct((1, n), jnp.int32),
    )(x, cbt_bf, b2, keys)


def _sc_gather(codebook, idx2d):
    n = idx2d.shape[1]
    d = codebook.shape[1]
    mesh = plsc.VectorSubcoreMesh(core_axis_name="c", subcore_axis_name="s")
    units = mesh.num_cores * mesh.num_subcores
    per = n // units

    tile = 64
    nbuf = 4
    ntiles = per // tile

    @functools.partial(
        pl.kernel,
        out_type=jax.ShapeDtypeStruct((n, d), codebook.dtype),
        mesh=mesh,
        scratch_types=[
            pltpu.VMEM((per,), jnp.int32),
            pltpu.VMEM((nbuf, tile, d), jnp.float32),
            pltpu.SemaphoreType.DMA,
            pltpu.SemaphoreType.DMA((nbuf,)),
            pltpu.SemaphoreType.DMA((nbuf,)),
        ],
    )
    def _gather_kernel(cb_hbm, i_hbm, o_hbm, iv, buf, isem, gsems, osems):
        c = jax.lax.axis_index("c")
        s = jax.lax.axis_index("s")
        base = (c * mesh.num_subcores + s) * per
        pltpu.async_copy(i_hbm.at[0, pl.ds(base, per)], iv, isem).wait()
        g_cps, o_cps = {}, {}
        for t in range(ntiles):
            b = t % nbuf
            if t >= nbuf:
                o_cps[t - nbuf].wait()
            g_cps[t] = pltpu.async_copy(
                cb_hbm.at[iv.at[pl.ds(t * tile, tile)]], buf.at[b], gsems.at[b]
            )
            if t >= 1:
                tp = t - 1
                g_cps[tp].wait()
                o_cps[tp] = pltpu.async_copy(
                    buf.at[tp % nbuf],
                    o_hbm.at[pl.ds(base + tp * tile, tile)],
                    osems.at[tp % nbuf],
                )
        tl = ntiles - 1
        g_cps[tl].wait()
        o_cps[tl] = pltpu.async_copy(
            buf.at[tl % nbuf], o_hbm.at[pl.ds(base + tl * tile, tile)],
            osems.at[tl % nbuf],
        )
        for t in range(max(0, ntiles - nbuf), ntiles):
            o_cps[t].wait()

    return _gather_kernel(codebook, idx2d)


_CHUNKS = 1


def kernel(x, codebook):
    n = x.shape[0]
    cbt = codebook.T
    k = codebook.shape[0]
    b2 = _codeword_norms(cbt)
    cbt_bf = cbt.astype(jnp.bfloat16)
    keys = jax.lax.bitcast_convert_type(
        jnp.int32(0x3F800000) + jnp.arange(k, dtype=jnp.int32), jnp.float32
    ).reshape(1, k)
    encs = [_assign(xc, cbt_bf, b2, keys) for xc in jnp.split(x, _CHUNKS)]
    qs = [_sc_gather(codebook, enc) for enc in encs]
    encoded = jnp.concatenate(encs, axis=1).reshape(n)
    quantized = jnp.concatenate(qs)
    return (encoded, quantized)
